# baseline (speedup 1.0000x reference)
import functools

import jax
import jax.numpy as jnp
from jax import lax
from jax.experimental import pallas as pl
from jax.experimental.pallas import tpu as pltpu
from jax.experimental.pallas import tpu_sc as plsc

NC = 2
NS = 16
NW = NC * NS
K = 128


def _msg_body(x_ref, w_ref, b_ref, o_ref):
    o_ref[...] = jnp.maximum(
        jnp.dot(x_ref[...], w_ref[...], preferred_element_type=jnp.float32)
        + b_ref[...], 0.0)


def _out_body(a_ref, d_ref, x_ref, w_ref, b_ref, o_ref):
    agg = a_ref[0] + a_ref[1]
    deg = d_ref[:, 0] + d_ref[:, 1]
    feats = jnp.where((deg > 0.0)[:, None], agg, x_ref[...])
    o_ref[...] = (
        jnp.dot(feats, w_ref[...], preferred_element_type=jnp.float32)
        + b_ref[...])


def _nblocks(E):
    nb = -(-(E // K) // NW)
    return nb + (-(nb - 1)) % 4


def _make_sc_scatter(N, D, E, NP):
    TB = E // K
    P = _nblocks(E)
    RS = -(-NP // NS)
    RS += (-RS) % 8
    LAST = NP - RS * (NS - 1)

    mesh = plsc.VectorSubcoreMesh(core_axis_name="c", subcore_axis_name="s")

    @functools.partial(
        pl.kernel,
        out_type=[jax.ShapeDtypeStruct((NC, NP, D), jnp.float32),
                  jax.ShapeDtypeStruct((NC * NP,), jnp.float32)],
        mesh=mesh,
        scratch_types=[
            pltpu.VMEM((4, 2, K), jnp.int32),
            pltpu.VMEM((K, D), jnp.float32),
            pltpu.VMEM((2, K, D), jnp.float32),
            pltpu.VMEM((K,), jnp.float32),
            pltpu.VMEM_SHARED((NP, D), jnp.float32),
            pltpu.VMEM_SHARED((NP,), jnp.float32),
            pltpu.SemaphoreType.DMA,
            pltpu.SemaphoreType.DMA,
            pltpu.SemaphoreType.DMA,
            pltpu.SemaphoreType.DMA,
            pltpu.SemaphoreType.DMA,
            pltpu.SemaphoreType.DMA,
            pltpu.SemaphoreType.DMA,
            pltpu.SemaphoreType.DMA,
            pltpu.SemaphoreType.DMA,
        ],
    )
    def sc_fn(m_hbm, edge_hbm, agg_out, deg_out,
              cidx, gbuf, rows2, ones_v, agg_sh, deg_sh,
              gsem, ssem0, ssem1, dsem0, dsem1,
              isem0, isem1, isem2, isem3):
        c = lax.axis_index("c")
        s = lax.axis_index("s")
        w = c * NS + s
        rows = (rows2.at[0], rows2.at[1])
        ssems = (ssem0, ssem1)
        dsems = (dsem0, dsem1)
        isems = (isem0, isem1, isem2, isem3)
        count = TB // NW + jnp.where(w < TB % NW, 1, 0)

        def _off(t):
            blk = jnp.minimum(w + NW * jnp.minimum(t, P - 1), TB - 1)
            return blk * K

        def load_idx(t, slot, sem):
            return pltpu.async_copy(
                edge_hbm.at[:, pl.ds(_off(t), K)], cidx.at[slot], sem)

        def gather(slot):
            return pltpu.async_copy(m_hbm.at[cidx.at[slot, 0]], gbuf, gsem)

        def scatter(slot, b):
            pltpu.async_copy(rows[b], agg_sh.at[cidx.at[slot, 1]],
                             ssems[b], add=True)
            pltpu.async_copy(ones_v, deg_sh.at[cidx.at[slot, 1]],
                             dsems[b], add=True)

        def wait_gather(slot):
            pltpu.make_async_copy(m_hbm.at[cidx.at[slot, 0]], gbuf,
                                  gsem).wait()

        def wait_scatter(slot, b):
            pltpu.make_async_copy(rows[b], agg_sh.at[cidx.at[slot, 1]],
                                  ssems[b]).wait()
            pltpu.make_async_copy(ones_v, deg_sh.at[cidx.at[slot, 1]],
                                  dsems[b]).wait()

        def wait_idx(t, slot, sem):
            pltpu.make_async_copy(
                edge_hbm.at[:, pl.ds(_off(t), K)], cidx.at[slot], sem).wait()

        def stage_rows(b):
            def row(r, _):
                for g in range(D // 16):
                    rows2[b, r, pl.ds(16 * g, 16)] = gbuf[r, pl.ds(16 * g, 16)]
                return 0
            lax.fori_loop(0, K, row, 0)

        for g in range(K // 16):
            ones_v[pl.ds(g * 16, 16)] = jnp.ones((16,), jnp.float32)

        def zrow_body(r, _):
            for g in range(D // 16):
                rows2[0, r, pl.ds(g * 16, 16)] = jnp.zeros((16,), jnp.float32)
            return 0
        lax.fori_loop(0, K, zrow_body, 0)

        def zero_span(base, n):
            full, rem = n // K, n % K
            for t in range(full):
                pltpu.sync_copy(rows2.at[0],
                                agg_sh.at[pl.ds(base + t * K, K)])
                pltpu.sync_copy(rows2.at[0, 0],
                                deg_sh.at[pl.ds(base + t * K, K)])
            if rem:
                pltpu.sync_copy(rows2.at[0, pl.ds(0, rem)],
                                agg_sh.at[pl.ds(base + full * K, rem)])
                pltpu.sync_copy(rows2.at[0, 0, pl.ds(0, rem)],
                                deg_sh.at[pl.ds(base + full * K, rem)])

        @pl.when(s < NS - 1)
        def _():
            zero_span(s * RS, RS)

        @pl.when(s == NS - 1)
        def _():
            zero_span((NS - 1) * RS, LAST)

        plsc.subcore_barrier()

        for q in range(3):
            load_idx(q, q, isems[q])
        wait_idx(0, 0, isems[0])
        gather(0)

        def pbody(j4, _):
            j = 4 * j4
            for q in range(4):
                t = j + q
                b = q % 2
                pb = (q + 1) % 2
                wait_gather(q)
                stage_rows(b)

                @pl.when(t < count)
                def _():
                    scatter(q, b)
                wait_idx(t + 1, (q + 1) % 4, isems[(q + 1) % 4])
                gather((q + 1) % 4)

                @pl.when(jnp.logical_and(t >= 1, t - 1 < count))
                def _():
                    wait_scatter((q + 3) % 4, pb)
                load_idx(t + 3, (q + 3) % 4, isems[(q + 3) % 4])
            return 0
        lax.fori_loop(0, P // 4, pbody, 0)

        wait_gather(0)

        @pl.when(P - 1 < count)
        def _():
            wait_scatter(3, (P - 1) % 2)
        wait_idx(P + 1, 1, isems[1])
        wait_idx(P + 2, 2, isems[2])

        plsc.subcore_barrier()

        def writeback(base, n):
            pltpu.sync_copy(agg_sh.at[pl.ds(base, n)],
                            agg_out.at[c, pl.ds(base, n)])
            full, rem = n // K, n % K
            for i in range(full):
                o = base + i * K
                pltpu.sync_copy(deg_sh.at[pl.ds(o, K)], ones_v)
                pltpu.sync_copy(ones_v, deg_out.at[pl.ds(c * NP + o, K)])
            if rem:
                o = base + full * K
                pltpu.sync_copy(deg_sh.at[pl.ds(o, rem)],
                                ones_v.at[pl.ds(0, rem)])
                pltpu.sync_copy(ones_v.at[pl.ds(0, rem)],
                                deg_out.at[pl.ds(c * NP + o, rem)])

        @pl.when(s < NS - 1)
        def _():
            writeback(s * RS, RS)

        @pl.when(s == NS - 1)
        def _():
            writeback((NS - 1) * RS, LAST)

    return sc_fn


def kernel(node_feats, edge_index, W_m, b_m, W_f, b_f):
    N, D = node_feats.shape
    E = edge_index.shape[1]
    NP = N

    BLK = 2048
    grid = (pl.cdiv(N, BLK),)

    M = pl.pallas_call(
        _msg_body,
        grid=grid,
        in_specs=[pl.BlockSpec((BLK, D), lambda i: (i, 0)),
                  pl.BlockSpec((D, D), lambda i: (0, 0)),
                  pl.BlockSpec((1, D), lambda i: (0, 0))],
        out_specs=pl.BlockSpec((BLK, D), lambda i: (i, 0)),
        out_shape=jax.ShapeDtypeStruct((N, D), jnp.float32),
    )(node_feats, W_m.T, b_m[None, :])

    agg_parts, deg_parts = _make_sc_scatter(N, D, E, NP)(M, edge_index)

    out = pl.pallas_call(
        _out_body,
        grid=grid,
        in_specs=[pl.BlockSpec((NC, BLK, D), lambda i: (0, i, 0)),
                  pl.BlockSpec((BLK, NC), lambda i: (i, 0)),
                  pl.BlockSpec((BLK, D), lambda i: (i, 0)),
                  pl.BlockSpec((D, D), lambda i: (0, 0)),
                  pl.BlockSpec((1, D), lambda i: (0, 0))],
        out_specs=pl.BlockSpec((BLK, D), lambda i: (i, 0)),
        out_shape=jax.ShapeDtypeStruct((N, D), jnp.float32),
    )(agg_parts, deg_parts.reshape(NC, N).T, node_feats, W_f.T, b_f[None, :])
    return out


# speedup vs baseline: 1.2440x; 1.2440x over previous
import functools

import jax
import jax.numpy as jnp
from jax import lax
from jax.experimental import pallas as pl
from jax.experimental.pallas import tpu as pltpu
from jax.experimental.pallas import tpu_sc as plsc

NC = 2
NS = 16
NW = NC * NS
K = 128


def _msg_body(x_ref, w_ref, b_ref, o_ref):
    o_ref[...] = jnp.maximum(
        jnp.dot(x_ref[...], w_ref[...], preferred_element_type=jnp.float32)
        + b_ref[...], 0.0)


def _out_body(a_ref, d_ref, x_ref, w_ref, b_ref, o_ref):
    agg = a_ref[0] + a_ref[1]
    deg = d_ref[:, 0] + d_ref[:, 1]
    feats = jnp.where((deg > 0.0)[:, None], agg, x_ref[...])
    o_ref[...] = (
        jnp.dot(feats, w_ref[...], preferred_element_type=jnp.float32)
        + b_ref[...])


def _nblocks(E):
    nb = -(-(E // K) // NW)
    return nb + (-(nb - 1)) % 4


def _make_sc_scatter(N, D, E, NP):
    TB = E // K
    P = _nblocks(E)
    RS = NP // NS
    DT = NP // NS

    mesh = plsc.VectorSubcoreMesh(core_axis_name="c", subcore_axis_name="s")

    @functools.partial(
        pl.kernel,
        out_type=[jax.ShapeDtypeStruct((NC, NP, D), jnp.float32),
                  jax.ShapeDtypeStruct((NC, NP), jnp.float32)],
        mesh=mesh,
        scratch_types=[
            pltpu.VMEM((4, 2, K), jnp.int32),
            pltpu.VMEM((2, K, D), jnp.float32),
            pltpu.VMEM((K,), jnp.float32),
            pltpu.VMEM((DT,), jnp.float32),
            pltpu.VMEM_SHARED((NP, D), jnp.float32),
            pltpu.VMEM_SHARED((NP,), jnp.float32),
            pltpu.SemaphoreType.DMA,
            pltpu.SemaphoreType.DMA,
            pltpu.SemaphoreType.DMA,
            pltpu.SemaphoreType.DMA,
            pltpu.SemaphoreType.DMA,
            pltpu.SemaphoreType.DMA,
            pltpu.SemaphoreType.DMA,
            pltpu.SemaphoreType.DMA,
            pltpu.SemaphoreType.DMA,
            pltpu.SemaphoreType.DMA,
        ],
    )
    def sc_fn(m_hbm, edge_hbm, agg_out, deg_out,
              cidx, rows2, ones_v, zdeg_v, agg_sh, deg_sh,
              gsem0, gsem1, ssem0, ssem1, dsem0, dsem1,
              isem0, isem1, isem2, isem3):
        c = lax.axis_index("c")
        s = lax.axis_index("s")
        w = c * NS + s
        rows = (rows2.at[0], rows2.at[1])
        gsems = (gsem0, gsem1)
        ssems = (ssem0, ssem1)
        dsems = (dsem0, dsem1)
        isems = (isem0, isem1, isem2, isem3)
        count = TB // NW + jnp.where(w < TB % NW, 1, 0)

        def _off(t):
            blk = jnp.minimum(w + NW * jnp.minimum(t, P - 1), TB - 1)
            return blk * K

        def load_idx(t, slot, sem):
            return pltpu.async_copy(
                edge_hbm.at[:, pl.ds(_off(t), K)], cidx.at[slot], sem)

        def gather(slot, b):
            return pltpu.async_copy(m_hbm.at[cidx.at[slot, 0]],
                                    rows[b], gsems[b])

        def scatter(slot, b):
            pltpu.async_copy(rows[b], agg_sh.at[cidx.at[slot, 1]],
                             ssems[b], add=True)
            pltpu.async_copy(ones_v, deg_sh.at[cidx.at[slot, 1]],
                             dsems[b], add=True)

        def wait_gather(slot, b):
            pltpu.make_async_copy(m_hbm.at[cidx.at[slot, 0]],
                                  rows[b], gsems[b]).wait()

        def wait_scatter(slot, b):
            pltpu.make_async_copy(rows[b], agg_sh.at[cidx.at[slot, 1]],
                                  ssems[b]).wait()
            pltpu.make_async_copy(ones_v, deg_sh.at[cidx.at[slot, 1]],
                                  dsems[b]).wait()

        def wait_idx(t, slot, sem):
            pltpu.make_async_copy(
                edge_hbm.at[:, pl.ds(_off(t), K)], cidx.at[slot], sem).wait()

        for g in range(K // 16):
            ones_v[pl.ds(g * 16, 16)] = jnp.ones((16,), jnp.float32)

        def zrow_body(r, _):
            for g in range(D // 16):
                rows2[0, r, pl.ds(g * 16, 16)] = jnp.zeros((16,), jnp.float32)
            return 0
        lax.fori_loop(0, K, zrow_body, 0)

        def zdeg_body(r, _):
            zdeg_v[pl.ds(r * 16, 16)] = jnp.zeros((16,), jnp.float32)
            return 0
        lax.fori_loop(0, DT // 16, zdeg_body, 0)

        for t in range(RS // K):
            pltpu.sync_copy(rows2.at[0], agg_sh.at[pl.ds(s * RS + t * K, K)])
        pltpu.sync_copy(zdeg_v, deg_sh.at[pl.ds(s * DT, DT)])

        plsc.subcore_barrier()

        for q in range(4):
            load_idx(q, q, isems[q])
        wait_idx(0, 0, isems[0])
        gather(0, 0)
        wait_idx(1, 1, isems[1])
        gather(1, 1)

        def pbody(j4, _):
            j = 4 * j4
            for h in range(2):
                q0, q1 = 2 * h, 2 * h + 1
                n0, n1 = (2 * h + 2) % 4, (2 * h + 3) % 4
                t0, t1 = j + q0, j + q1
                wait_gather(q0, 0)

                @pl.when(t0 < count)
                def _():
                    scatter(q0, 0)
                wait_gather(q1, 1)

                @pl.when(t1 < count)
                def _():
                    scatter(q1, 1)

                @pl.when(t0 < count)
                def _():
                    wait_scatter(q0, 0)
                load_idx(t0 + 4, q0, isems[q0])
                wait_idx(j + n0, n0, isems[n0])
                gather(n0, 0)

                @pl.when(t1 < count)
                def _():
                    wait_scatter(q1, 1)
                load_idx(t1 + 4, q1, isems[q1])
                wait_idx(j + n1, n1, isems[n1])
                gather(n1, 1)
            return 0
        lax.fori_loop(0, (P - 1) // 4, pbody, 0)

        wait_gather(0, 0)

        @pl.when(P - 1 < count)
        def _():
            scatter(0, 0)
        wait_gather(1, 1)
        wait_idx(P - 1, 2, isems[2])
        wait_idx(P - 1, 3, isems[3])

        @pl.when(P - 1 < count)
        def _():
            wait_scatter(0, 0)

        plsc.subcore_barrier()

        sl = pl.ds(s * RS, RS)
        pltpu.sync_copy(agg_sh.at[sl], agg_out.at[c, sl])
        pltpu.sync_copy(deg_sh.at[pl.ds(s * DT, DT)],
                        deg_out.at[c, pl.ds(s * DT, DT)])

    return sc_fn


def kernel(node_feats, edge_index, W_m, b_m, W_f, b_f):
    N, D = node_feats.shape
    E = edge_index.shape[1]
    NP = ((N + NW * 16 - 1) // (NW * 16)) * (NW * 16)

    BLK = 2048
    grid = (pl.cdiv(N, BLK),)

    M = pl.pallas_call(
        _msg_body,
        grid=grid,
        in_specs=[pl.BlockSpec((BLK, D), lambda i: (i, 0)),
                  pl.BlockSpec((D, D), lambda i: (0, 0)),
                  pl.BlockSpec((1, D), lambda i: (0, 0))],
        out_specs=pl.BlockSpec((BLK, D), lambda i: (i, 0)),
        out_shape=jax.ShapeDtypeStruct((N, D), jnp.float32),
    )(node_feats, W_m.T, b_m[None, :])

    agg_parts, deg_parts = _make_sc_scatter(N, D, E, NP)(M, edge_index)

    out = pl.pallas_call(
        _out_body,
        grid=grid,
        in_specs=[pl.BlockSpec((NC, BLK, D), lambda i: (0, i, 0)),
                  pl.BlockSpec((BLK, NC), lambda i: (i, 0)),
                  pl.BlockSpec((BLK, D), lambda i: (i, 0)),
                  pl.BlockSpec((D, D), lambda i: (0, 0)),
                  pl.BlockSpec((1, D), lambda i: (0, 0))],
        out_specs=pl.BlockSpec((BLK, D), lambda i: (i, 0)),
        out_shape=jax.ShapeDtypeStruct((N, D), jnp.float32),
    )(agg_parts, deg_parts[:, :N].T, node_feats, W_f.T, b_f[None, :])
    return out
